# baseline (speedup 1.0000x reference)
import functools

import jax
import jax.numpy as jnp
from jax import lax
from jax.experimental import pallas as pl
from jax.experimental.pallas import tpu as pltpu
from jax.experimental.pallas import tpu_sc as plsc

NUM_CLASSES = 1000000
DIM = 64
BATCH = 1024
SEQ = 200
BL = BATCH * SEQ

NUM_WORKERS = 32
ROWS_PER_WORKER = BL // NUM_WORKERS
CHUNK = 128
NUM_CHUNKS = ROWS_PER_WORKER // CHUNK

_INV2PI = 0.15915494309189535
_MAGIC = 1.5 * 2.0**23
_C1 = 6.28125
_C2 = 6.283185307179586 - 6.28125

_SIN_C = (9.9999970e-01, -1.6666578e-01, 8.3325580e-03, -1.9812575e-04,
          2.7040512e-06, -2.0534245e-08)
_COS_C = (1.0000000e+00, -4.9999991e-01, 4.1666523e-02, -1.3887971e-03,
          2.4773424e-05, -2.7113370e-07, 1.7369117e-09)


def _sincos(p):
    k = (p * _INV2PI + _MAGIC) - _MAGIC
    r = (p - k * _C1) - k * _C2
    t = r * r
    s = _SIN_C[5]
    for c in (_SIN_C[4], _SIN_C[3], _SIN_C[2], _SIN_C[1], _SIN_C[0]):
        s = s * t + c
    s = s * r
    c_ = _COS_C[6]
    for c in (_COS_C[5], _COS_C[4], _COS_C[3], _COS_C[2], _COS_C[1], _COS_C[0]):
        c_ = c_ * t + c
    return s, c_


def _body(words_hbm, amp_hbm, freq_hbm, out_hbm,
          idx_v, amp_v, freq_v, out_v, sem_a, sem_f):
    nc = 2
    wid = lax.axis_index("s") * nc + lax.axis_index("c")
    worker_base = wid * ROWS_PER_WORKER

    def chunk_body(g, carry):
        base = worker_base + g * CHUNK
        pltpu.sync_copy(words_hbm.at[pl.ds(base, CHUNK)], idx_v)
        cp_a = pltpu.async_copy(amp_hbm.at[idx_v], amp_v, sem_a)
        cp_f = pltpu.async_copy(freq_hbm.at[idx_v], freq_v, sem_f)
        cp_a.wait()
        cp_f.wait()

        def row_body(r, carry2):
            pos_i = (base + r) % SEQ + 1
            posv = jnp.full((16,), pos_i.astype(jnp.float32), jnp.float32)
            for j in range(DIM // 16):
                a = amp_v[r, pl.ds(16 * j, 16)]
                f = freq_v[r, pl.ds(16 * j, 16)]
                s, c = _sincos(f * posv)
                out_v[r, pl.ds(16 * j, 16)] = a * c
                out_v[r, pl.ds(DIM + 16 * j, 16)] = a * s
            return carry2

        lax.fori_loop(0, CHUNK, row_body, 0)
        pltpu.sync_copy(out_v, out_hbm.at[pl.ds(base, CHUNK)])
        return carry

    lax.fori_loop(0, NUM_CHUNKS, chunk_body, 0)


@functools.partial(jax.jit, static_argnames=())
def _run(words_flat, amp_table, freq_table):
    mesh = plsc.VectorSubcoreMesh(core_axis_name="c", subcore_axis_name="s")
    call = functools.partial(
        pl.kernel,
        mesh=mesh,
        compiler_params=pltpu.CompilerParams(use_tc_tiling_on_sc=False),
        out_type=jax.ShapeDtypeStruct((BL, 2 * DIM), jnp.float32),
        scratch_types=[
            pltpu.VMEM((CHUNK,), jnp.int32),
            pltpu.VMEM((CHUNK, DIM), jnp.float32),
            pltpu.VMEM((CHUNK, DIM), jnp.float32),
            pltpu.VMEM((CHUNK, 2 * DIM), jnp.float32),
            pltpu.SemaphoreType.DMA,
            pltpu.SemaphoreType.DMA,
        ],
    )(_body)
    return call(words_flat, amp_table, freq_table)


def kernel(words, amp_table, freq_table):
    words_flat = words.reshape(BL).astype(jnp.int32)
    out = _run(words_flat, amp_table, freq_table)
    return out.reshape(BATCH, SEQ, 2 * DIM)


# speedup vs baseline: 1.0496x; 1.0496x over previous
import functools

import jax
import jax.numpy as jnp
from jax import lax
from jax.experimental import pallas as pl
from jax.experimental.pallas import tpu as pltpu
from jax.experimental.pallas import tpu_sc as plsc

NUM_CLASSES = 1000000
DIM = 64
BATCH = 1024
SEQ = 200
BL = BATCH * SEQ

NUM_WORKERS = 32
ROWS_PER_WORKER = BL // NUM_WORKERS
CHUNK = 128
NUM_CHUNKS = ROWS_PER_WORKER // CHUNK
NUM_PAIRS = NUM_CHUNKS // 2

_INV2PI = 0.15915494309189535
_MAGIC = 1.5 * 2.0**23
_C1 = 6.28125
_C2 = 6.283185307179586 - 6.28125

_SIN_C = (9.9999970e-01, -1.6666578e-01, 8.3325580e-03, -1.9812575e-04,
          2.7040512e-06, -2.0534245e-08)
_COS_C = (1.0000000e+00, -4.9999991e-01, 4.1666523e-02, -1.3887971e-03,
          2.4773424e-05, -2.7113370e-07, 1.7369117e-09)


def _sincos(p):
    k = (p * _INV2PI + _MAGIC) - _MAGIC
    r = (p - k * _C1) - k * _C2
    t = r * r
    s = _SIN_C[5]
    for c in (_SIN_C[4], _SIN_C[3], _SIN_C[2], _SIN_C[1], _SIN_C[0]):
        s = s * t + c
    s = s * r
    c_ = _COS_C[6]
    for c in (_COS_C[5], _COS_C[4], _COS_C[3], _COS_C[2], _COS_C[1], _COS_C[0]):
        c_ = c_ * t + c
    return s, c_


def _body(words_hbm, amp_hbm, freq_hbm, out_hbm,
          idx0, idx1, bufA0, bufF0, bufA1, bufF1, ob0, ob1,
          gsem0, gsem1, wsem0, wsem1):
    nc = 2
    wid = lax.axis_index("s") * nc + lax.axis_index("c")
    worker_base = wid * ROWS_PER_WORKER

    def compute(bufA, bufF, ob, base):
        def row_body(r, carry):
            pos_i = (base + r) % SEQ + 1
            posv = jnp.full((16,), pos_i.astype(jnp.float32), jnp.float32)
            for j in range(DIM // 16):
                a = bufA[r, pl.ds(16 * j, 16)]
                f = bufF[r, pl.ds(16 * j, 16)]
                s, c = _sincos(f * posv)
                ob[r, pl.ds(16 * j, 16)] = a * c
                ob[r, pl.ds(DIM + 16 * j, 16)] = a * s
            return carry

        lax.fori_loop(0, CHUNK, row_body, 0)

    def start_gather(g, idx, bufA, bufF, gsem):
        base = worker_base + g * CHUNK
        pltpu.sync_copy(words_hbm.at[pl.ds(base, CHUNK)], idx)
        pltpu.make_async_copy(amp_hbm.at[idx], bufA, gsem).start()
        pltpu.make_async_copy(freq_hbm.at[idx], bufF, gsem).start()

    def wait_gather(idx, bufA, bufF, gsem):
        pltpu.make_async_copy(amp_hbm.at[idx], bufA, gsem).wait()
        pltpu.make_async_copy(freq_hbm.at[idx], bufF, gsem).wait()

    start_gather(0, idx0, bufA0, bufF0, gsem0)

    def pair_body(i, carry):
        g0 = 2 * i
        g1 = g0 + 1
        base0 = worker_base + g0 * CHUNK
        base1 = worker_base + g1 * CHUNK

        start_gather(g1, idx1, bufA1, bufF1, gsem1)

        wait_gather(idx0, bufA0, bufF0, gsem0)

        @pl.when(i > 0)
        def _():
            pltpu.make_async_copy(ob0, out_hbm.at[pl.ds(base0, CHUNK)], wsem0).wait()

        compute(bufA0, bufF0, ob0, base0)
        pltpu.make_async_copy(ob0, out_hbm.at[pl.ds(base0, CHUNK)], wsem0).start()

        @pl.when(i < NUM_PAIRS - 1)
        def _():
            start_gather(g0 + 2, idx0, bufA0, bufF0, gsem0)

        wait_gather(idx1, bufA1, bufF1, gsem1)

        @pl.when(i > 0)
        def _():
            pltpu.make_async_copy(ob1, out_hbm.at[pl.ds(base1, CHUNK)], wsem1).wait()

        compute(bufA1, bufF1, ob1, base1)
        pltpu.make_async_copy(ob1, out_hbm.at[pl.ds(base1, CHUNK)], wsem1).start()
        return carry

    lax.fori_loop(0, NUM_PAIRS, pair_body, 0)

    last0 = worker_base + (NUM_CHUNKS - 2) * CHUNK
    last1 = worker_base + (NUM_CHUNKS - 1) * CHUNK
    pltpu.make_async_copy(ob0, out_hbm.at[pl.ds(last0, CHUNK)], wsem0).wait()
    pltpu.make_async_copy(ob1, out_hbm.at[pl.ds(last1, CHUNK)], wsem1).wait()


@jax.jit
def _run(words_flat, amp_table, freq_table):
    mesh = plsc.VectorSubcoreMesh(core_axis_name="c", subcore_axis_name="s")
    call = functools.partial(
        pl.kernel,
        mesh=mesh,
        compiler_params=pltpu.CompilerParams(use_tc_tiling_on_sc=False),
        out_type=jax.ShapeDtypeStruct((BL, 2 * DIM), jnp.float32),
        scratch_types=[
            pltpu.VMEM((CHUNK,), jnp.int32),
            pltpu.VMEM((CHUNK,), jnp.int32),
            pltpu.VMEM((CHUNK, DIM), jnp.float32),
            pltpu.VMEM((CHUNK, DIM), jnp.float32),
            pltpu.VMEM((CHUNK, DIM), jnp.float32),
            pltpu.VMEM((CHUNK, DIM), jnp.float32),
            pltpu.VMEM((CHUNK, 2 * DIM), jnp.float32),
            pltpu.VMEM((CHUNK, 2 * DIM), jnp.float32),
            pltpu.SemaphoreType.DMA,
            pltpu.SemaphoreType.DMA,
            pltpu.SemaphoreType.DMA,
            pltpu.SemaphoreType.DMA,
        ],
    )(_body)
    return call(words_flat, amp_table, freq_table)


def kernel(words, amp_table, freq_table):
    words_flat = words.reshape(BL).astype(jnp.int32)
    out = _run(words_flat, amp_table, freq_table)
    return out.reshape(BATCH, SEQ, 2 * DIM)


# speedup vs baseline: 1.1694x; 1.1141x over previous
import functools

import jax
import jax.numpy as jnp
from jax import lax
from jax.experimental import pallas as pl
from jax.experimental.pallas import tpu as pltpu
from jax.experimental.pallas import tpu_sc as plsc

NUM_CLASSES = 1000000
DIM = 64
BATCH = 1024
SEQ = 200
BL = BATCH * SEQ

NUM_WORKERS = 32
ROWS_PER_WORKER = BL // NUM_WORKERS
CHUNK = 128
NUM_CHUNKS = ROWS_PER_WORKER // CHUNK
NUM_PAIRS = NUM_CHUNKS // 2

_INV2PI = 0.15915494309189535
_MAGIC = 1.5 * 2.0**23
_C1 = 6.28125
_C2 = 6.283185307179586 - 6.28125

_S0, _S1, _S2, _S3 = (9.9945015e-01, -1.6583844e-01, 7.9985755e-03,
                      -1.4774044e-04)
_D0, _D1, _D2, _D3, _D4 = (9.9997109e-01, -4.9983761e-01, 4.1522305e-02,
                           -1.3441069e-03, 1.9065215e-05)


def _sincos(p):
    k = (p * _INV2PI + _MAGIC) - _MAGIC
    r = (p - k * _C1) - k * _C2
    t = r * r
    t2 = t * t
    s = ((_S0 + _S1 * t) + t2 * (_S2 + _S3 * t)) * r
    c_ = (_D0 + _D1 * t) + t2 * ((_D2 + _D3 * t) + t2 * _D4)
    return s, c_


def _body(words_hbm, amp_hbm, freq_hbm, out_hbm,
          idx0, idx1, bufA0, bufF0, bufA1, bufF1, ob0, ob1,
          gsem0, gsem1, wsem0, wsem1):
    nc = 2
    wid = lax.axis_index("s") * nc + lax.axis_index("c")
    worker_base = wid * ROWS_PER_WORKER

    def compute(bufA, bufF, ob, base):
        pos_init = base % SEQ + 1
        unroll = 4

        def row_body(i, pos_i):
            r0 = i * unroll
            for u in range(unroll):
                r = r0 + u
                posv = jnp.full((16,), pos_i.astype(jnp.float32), jnp.float32)
                for j in range(DIM // 16):
                    a = bufA[r, pl.ds(16 * j, 16)]
                    f = bufF[r, pl.ds(16 * j, 16)]
                    s, c = _sincos(f * posv)
                    ob[r, pl.ds(16 * j, 16)] = a * c
                    ob[r, pl.ds(DIM + 16 * j, 16)] = a * s
                pos_i = jnp.where(pos_i == SEQ, 1, pos_i + 1)
            return pos_i

        lax.fori_loop(0, CHUNK // unroll, row_body, pos_init)

    def start_gather(g, idx, bufA, bufF, gsem):
        base = worker_base + g * CHUNK
        pltpu.sync_copy(words_hbm.at[pl.ds(base, CHUNK)], idx)
        pltpu.make_async_copy(amp_hbm.at[idx], bufA, gsem).start()
        pltpu.make_async_copy(freq_hbm.at[idx], bufF, gsem).start()

    def wait_gather(idx, bufA, bufF, gsem):
        pltpu.make_async_copy(amp_hbm.at[idx], bufA, gsem).wait()
        pltpu.make_async_copy(freq_hbm.at[idx], bufF, gsem).wait()

    start_gather(0, idx0, bufA0, bufF0, gsem0)

    def pair_body(i, carry):
        g0 = 2 * i
        g1 = g0 + 1
        base0 = worker_base + g0 * CHUNK
        base1 = worker_base + g1 * CHUNK

        start_gather(g1, idx1, bufA1, bufF1, gsem1)

        wait_gather(idx0, bufA0, bufF0, gsem0)

        @pl.when(i > 0)
        def _():
            pltpu.make_async_copy(ob0, out_hbm.at[pl.ds(base0, CHUNK)], wsem0).wait()

        compute(bufA0, bufF0, ob0, base0)
        pltpu.make_async_copy(ob0, out_hbm.at[pl.ds(base0, CHUNK)], wsem0).start()

        @pl.when(i < NUM_PAIRS - 1)
        def _():
            start_gather(g0 + 2, idx0, bufA0, bufF0, gsem0)

        wait_gather(idx1, bufA1, bufF1, gsem1)

        @pl.when(i > 0)
        def _():
            pltpu.make_async_copy(ob1, out_hbm.at[pl.ds(base1, CHUNK)], wsem1).wait()

        compute(bufA1, bufF1, ob1, base1)
        pltpu.make_async_copy(ob1, out_hbm.at[pl.ds(base1, CHUNK)], wsem1).start()
        return carry

    lax.fori_loop(0, NUM_PAIRS, pair_body, 0)

    last0 = worker_base + (NUM_CHUNKS - 2) * CHUNK
    last1 = worker_base + (NUM_CHUNKS - 1) * CHUNK
    pltpu.make_async_copy(ob0, out_hbm.at[pl.ds(last0, CHUNK)], wsem0).wait()
    pltpu.make_async_copy(ob1, out_hbm.at[pl.ds(last1, CHUNK)], wsem1).wait()


@jax.jit
def _run(words_flat, amp_table, freq_table):
    mesh = plsc.VectorSubcoreMesh(core_axis_name="c", subcore_axis_name="s")
    call = functools.partial(
        pl.kernel,
        mesh=mesh,
        compiler_params=pltpu.CompilerParams(use_tc_tiling_on_sc=False),
        out_type=jax.ShapeDtypeStruct((BL, 2 * DIM), jnp.float32),
        scratch_types=[
            pltpu.VMEM((CHUNK,), jnp.int32),
            pltpu.VMEM((CHUNK,), jnp.int32),
            pltpu.VMEM((CHUNK, DIM), jnp.float32),
            pltpu.VMEM((CHUNK, DIM), jnp.float32),
            pltpu.VMEM((CHUNK, DIM), jnp.float32),
            pltpu.VMEM((CHUNK, DIM), jnp.float32),
            pltpu.VMEM((CHUNK, 2 * DIM), jnp.float32),
            pltpu.VMEM((CHUNK, 2 * DIM), jnp.float32),
            pltpu.SemaphoreType.DMA,
            pltpu.SemaphoreType.DMA,
            pltpu.SemaphoreType.DMA,
            pltpu.SemaphoreType.DMA,
        ],
    )(_body)
    return call(words_flat, amp_table, freq_table)


def kernel(words, amp_table, freq_table):
    words_flat = words.reshape(BL).astype(jnp.int32)
    out = _run(words_flat, amp_table, freq_table)
    return out.reshape(BATCH, SEQ, 2 * DIM)


# speedup vs baseline: 1.4910x; 1.2750x over previous
import functools

import jax
import jax.numpy as jnp
from jax import lax
from jax.experimental import pallas as pl
from jax.experimental.pallas import tpu as pltpu
from jax.experimental.pallas import tpu_sc as plsc

NUM_CLASSES = 1000000
DIM = 64
BATCH = 1024
SEQ = 200
BL = BATCH * SEQ

NUM_WORKERS = 32
ROWS_PER_WORKER = BL // NUM_WORKERS
CHUNK = 128
NUM_CHUNKS = ROWS_PER_WORKER // CHUNK
NUM_PAIRS = NUM_CHUNKS // 2

_INV2PI = 0.15915494309189535
_MAGIC = 1.5 * 2.0**23
_C1 = 6.28125
_C2 = 6.283185307179586 - 6.28125

_S0, _S1, _S2, _S3 = (9.9945015e-01, -1.6583844e-01, 7.9985755e-03,
                      -1.4774044e-04)
_D0, _D1, _D2, _D3, _D4 = (9.9997109e-01, -4.9983761e-01, 4.1522305e-02,
                           -1.3441069e-03, 1.9065215e-05)


def _sincos(p):
    k = (p * _INV2PI + _MAGIC) - _MAGIC
    r = (p - k * _C1) - k * _C2
    t = r * r
    t2 = t * t
    s = ((_S0 + _S1 * t) + t2 * (_S2 + _S3 * t)) * r
    c_ = (_D0 + _D1 * t) + t2 * ((_D2 + _D3 * t) + t2 * _D4)
    return s, c_


def _body(words_hbm, amp_hbm, freq_hbm, out_hbm,
          idx0, idx1, bufA0, bufF0, bufA1, bufF1, ob0, ob1,
          gsem0, gsem1, wsem0, wsem1):
    nc = 2
    wid = lax.axis_index("s") * nc + lax.axis_index("c")
    worker_base = wid * ROWS_PER_WORKER

    def compute(bufA, bufF, ob, base):
        pos_init = base % SEQ + 1

        @plsc.parallel_loop(0, CHUNK, carry=pos_init, unroll=8)
        def row_body(r, pos_i):
            posv = jnp.full((16,), pos_i.astype(jnp.float32), jnp.float32)
            for j in range(DIM // 16):
                a = bufA[r, pl.ds(16 * j, 16)]
                f = bufF[r, pl.ds(16 * j, 16)]
                s, c = _sincos(f * posv)
                ob[r, pl.ds(16 * j, 16)] = a * c
                ob[r, pl.ds(DIM + 16 * j, 16)] = a * s
            return jnp.where(pos_i == SEQ, 1, pos_i + 1)

    def start_gather(g, idx, bufA, bufF, gsem):
        base = worker_base + g * CHUNK
        pltpu.sync_copy(words_hbm.at[pl.ds(base, CHUNK)], idx)
        pltpu.make_async_copy(amp_hbm.at[idx], bufA, gsem).start()
        pltpu.make_async_copy(freq_hbm.at[idx], bufF, gsem).start()

    def wait_gather(idx, bufA, bufF, gsem):
        pltpu.make_async_copy(amp_hbm.at[idx], bufA, gsem).wait()
        pltpu.make_async_copy(freq_hbm.at[idx], bufF, gsem).wait()

    start_gather(0, idx0, bufA0, bufF0, gsem0)

    def pair_body(i, carry):
        g0 = 2 * i
        g1 = g0 + 1
        base0 = worker_base + g0 * CHUNK
        base1 = worker_base + g1 * CHUNK

        start_gather(g1, idx1, bufA1, bufF1, gsem1)

        wait_gather(idx0, bufA0, bufF0, gsem0)

        @pl.when(i > 0)
        def _():
            pltpu.make_async_copy(ob0, out_hbm.at[pl.ds(base0, CHUNK)], wsem0).wait()

        compute(bufA0, bufF0, ob0, base0)
        pltpu.make_async_copy(ob0, out_hbm.at[pl.ds(base0, CHUNK)], wsem0).start()

        @pl.when(i < NUM_PAIRS - 1)
        def _():
            start_gather(g0 + 2, idx0, bufA0, bufF0, gsem0)

        wait_gather(idx1, bufA1, bufF1, gsem1)

        @pl.when(i > 0)
        def _():
            pltpu.make_async_copy(ob1, out_hbm.at[pl.ds(base1, CHUNK)], wsem1).wait()

        compute(bufA1, bufF1, ob1, base1)
        pltpu.make_async_copy(ob1, out_hbm.at[pl.ds(base1, CHUNK)], wsem1).start()
        return carry

    lax.fori_loop(0, NUM_PAIRS, pair_body, 0)

    last0 = worker_base + (NUM_CHUNKS - 2) * CHUNK
    last1 = worker_base + (NUM_CHUNKS - 1) * CHUNK
    pltpu.make_async_copy(ob0, out_hbm.at[pl.ds(last0, CHUNK)], wsem0).wait()
    pltpu.make_async_copy(ob1, out_hbm.at[pl.ds(last1, CHUNK)], wsem1).wait()


@jax.jit
def _run(words_flat, amp_table, freq_table):
    mesh = plsc.VectorSubcoreMesh(core_axis_name="c", subcore_axis_name="s")
    call = functools.partial(
        pl.kernel,
        mesh=mesh,
        compiler_params=pltpu.CompilerParams(use_tc_tiling_on_sc=False),
        out_type=jax.ShapeDtypeStruct((BL, 2 * DIM), jnp.float32),
        scratch_types=[
            pltpu.VMEM((CHUNK,), jnp.int32),
            pltpu.VMEM((CHUNK,), jnp.int32),
            pltpu.VMEM((CHUNK, DIM), jnp.float32),
            pltpu.VMEM((CHUNK, DIM), jnp.float32),
            pltpu.VMEM((CHUNK, DIM), jnp.float32),
            pltpu.VMEM((CHUNK, DIM), jnp.float32),
            pltpu.VMEM((CHUNK, 2 * DIM), jnp.float32),
            pltpu.VMEM((CHUNK, 2 * DIM), jnp.float32),
            pltpu.SemaphoreType.DMA,
            pltpu.SemaphoreType.DMA,
            pltpu.SemaphoreType.DMA,
            pltpu.SemaphoreType.DMA,
        ],
    )(_body)
    return call(words_flat, amp_table, freq_table)


def kernel(words, amp_table, freq_table):
    words_flat = words.reshape(BL).astype(jnp.int32)
    out = _run(words_flat, amp_table, freq_table)
    return out.reshape(BATCH, SEQ, 2 * DIM)


# speedup vs baseline: 1.5107x; 1.0132x over previous
import functools

import jax
import jax.numpy as jnp
from jax import lax
from jax.experimental import pallas as pl
from jax.experimental.pallas import tpu as pltpu
from jax.experimental.pallas import tpu_sc as plsc

NUM_CLASSES = 1000000
DIM = 64
BATCH = 1024
SEQ = 200
BL = BATCH * SEQ

NUM_WORKERS = 32
ROWS_PER_WORKER = BL // NUM_WORKERS
CHUNK = 128
NUM_CHUNKS = ROWS_PER_WORKER // CHUNK
NUM_PAIRS = NUM_CHUNKS // 2

_INV2PI = 0.15915494309189535
_MAGIC = 1.5 * 2.0**23
_C1 = 6.28125
_C2 = 6.283185307179586 - 6.28125

_S0, _S1, _S2, _S3 = (9.9945015e-01, -1.6583844e-01, 7.9985755e-03,
                      -1.4774044e-04)
_D0, _D1, _D2, _D3, _D4 = (9.9997109e-01, -4.9983761e-01, 4.1522305e-02,
                           -1.3441069e-03, 1.9065215e-05)


def _sincos(p):
    k = (p * _INV2PI + _MAGIC) - _MAGIC
    r = (p - k * _C1) - k * _C2
    t = r * r
    t2 = t * t
    s = ((_S0 + _S1 * t) + t2 * (_S2 + _S3 * t)) * r
    c_ = (_D0 + _D1 * t) + t2 * ((_D2 + _D3 * t) + t2 * _D4)
    return s, c_


def _body(words_hbm, amp_hbm, freq_hbm, out_hbm,
          idx0, idx1, bufA0, bufF0, bufA1, bufF1, ob0, ob1,
          gsem0, gsem1, wsem0, wsem1):
    nc = 2
    wid = lax.axis_index("s") * nc + lax.axis_index("c")
    chunk0 = wid * NUM_CHUNKS

    def compute(bufA, bufF, ob, l):
        posv = jnp.full((16,), (l + 1).astype(jnp.float32), jnp.float32)

        @plsc.parallel_loop(0, CHUNK, unroll=8)
        def row_body(r):
            for j in range(DIM // 16):
                a = bufA[r, pl.ds(16 * j, 16)]
                f = bufF[r, pl.ds(16 * j, 16)]
                s, c = _sincos(f * posv)
                ob[r, pl.ds(16 * j, 16)] = a * c
                ob[r, pl.ds(DIM + 16 * j, 16)] = a * s

    def start_gather(g, idx, bufA, bufF, gsem):
        pltpu.sync_copy(words_hbm.at[pl.ds(g * CHUNK, CHUNK)], idx)
        pltpu.make_async_copy(amp_hbm.at[idx], bufA, gsem).start()
        pltpu.make_async_copy(freq_hbm.at[idx], bufF, gsem).start()

    def wait_gather(idx, bufA, bufF, gsem):
        pltpu.make_async_copy(amp_hbm.at[idx], bufA, gsem).wait()
        pltpu.make_async_copy(freq_hbm.at[idx], bufF, gsem).wait()

    def out_block(g):
        q0 = g * CHUNK
        l = q0 // BATCH
        b0 = q0 % BATCH
        return out_hbm.at[pl.ds(b0, CHUNK), l], l

    start_gather(chunk0, idx0, bufA0, bufF0, gsem0)

    def pair_body(i, carry):
        g0 = chunk0 + 2 * i
        g1 = g0 + 1
        dst0, l0 = out_block(g0)
        dst1, l1 = out_block(g1)

        start_gather(g1, idx1, bufA1, bufF1, gsem1)

        wait_gather(idx0, bufA0, bufF0, gsem0)

        @pl.when(i > 0)
        def _():
            pltpu.make_async_copy(ob0, dst0, wsem0).wait()

        compute(bufA0, bufF0, ob0, l0)
        pltpu.make_async_copy(ob0, dst0, wsem0).start()

        @pl.when(i < NUM_PAIRS - 1)
        def _():
            start_gather(g0 + 2, idx0, bufA0, bufF0, gsem0)

        wait_gather(idx1, bufA1, bufF1, gsem1)

        @pl.when(i > 0)
        def _():
            pltpu.make_async_copy(ob1, dst1, wsem1).wait()

        compute(bufA1, bufF1, ob1, l1)
        pltpu.make_async_copy(ob1, dst1, wsem1).start()
        return carry

    lax.fori_loop(0, NUM_PAIRS, pair_body, 0)

    dstl0, _ = out_block(chunk0 + NUM_CHUNKS - 2)
    dstl1, _ = out_block(chunk0 + NUM_CHUNKS - 1)
    pltpu.make_async_copy(ob0, dstl0, wsem0).wait()
    pltpu.make_async_copy(ob1, dstl1, wsem1).wait()


@jax.jit
def _run(words_flat, amp_table, freq_table):
    mesh = plsc.VectorSubcoreMesh(core_axis_name="c", subcore_axis_name="s")
    call = functools.partial(
        pl.kernel,
        mesh=mesh,
        compiler_params=pltpu.CompilerParams(use_tc_tiling_on_sc=False),
        out_type=jax.ShapeDtypeStruct((BATCH, SEQ, 2 * DIM), jnp.float32),
        scratch_types=[
            pltpu.VMEM((CHUNK,), jnp.int32),
            pltpu.VMEM((CHUNK,), jnp.int32),
            pltpu.VMEM((CHUNK, DIM), jnp.float32),
            pltpu.VMEM((CHUNK, DIM), jnp.float32),
            pltpu.VMEM((CHUNK, DIM), jnp.float32),
            pltpu.VMEM((CHUNK, DIM), jnp.float32),
            pltpu.VMEM((CHUNK, 2 * DIM), jnp.float32),
            pltpu.VMEM((CHUNK, 2 * DIM), jnp.float32),
            pltpu.SemaphoreType.DMA,
            pltpu.SemaphoreType.DMA,
            pltpu.SemaphoreType.DMA,
            pltpu.SemaphoreType.DMA,
        ],
    )(_body)
    return call(words_flat, amp_table, freq_table)


def kernel(words, amp_table, freq_table):
    words_flat = words.T.reshape(BL).astype(jnp.int32)
    return _run(words_flat, amp_table, freq_table)


# speedup vs baseline: 1.7777x; 1.1767x over previous
import functools

import jax
import jax.numpy as jnp
from jax import lax
from jax.experimental import pallas as pl
from jax.experimental.pallas import tpu as pltpu
from jax.experimental.pallas import tpu_sc as plsc

NUM_CLASSES = 1000000
DIM = 64
BATCH = 1024
SEQ = 200
BL = BATCH * SEQ

NUM_WORKERS = 32
ROWS_PER_WORKER = BL // NUM_WORKERS
CHUNK = 128
NUM_CHUNKS = ROWS_PER_WORKER // CHUNK
NUM_PAIRS = NUM_CHUNKS // 2

_INV2PI = 0.15915494309189535
_MAGIC = 1.5 * 2.0**23
_C1 = 6.28125
_C2 = 6.283185307179586 - 6.28125

_S0, _S1, _S2, _S3 = (9.9945015e-01, -1.6583844e-01, 7.9985755e-03,
                      -1.4774044e-04)
_D0, _D1, _D2, _D3, _D4 = (9.9997109e-01, -4.9983761e-01, 4.1522305e-02,
                           -1.3441069e-03, 1.9065215e-05)


def _sincos(p):
    k = (p * _INV2PI + _MAGIC) - _MAGIC
    r = (p - k * _C1) - k * _C2
    t = r * r
    t2 = t * t
    s = ((_S0 + _S1 * t) + t2 * (_S2 + _S3 * t)) * r
    c_ = (_D0 + _D1 * t) + t2 * ((_D2 + _D3 * t) + t2 * _D4)
    return s, c_


def _body(words_hbm, tab_hbm, out_hbm,
          idx0, idx1, buf0, buf1, ob0, ob1,
          gsem0, gsem1, wsem0, wsem1):
    nc = 2
    wid = lax.axis_index("s") * nc + lax.axis_index("c")
    chunk0 = wid * NUM_CHUNKS

    def compute(buf, ob, l):
        posv = jnp.full((16,), (l + 1).astype(jnp.float32), jnp.float32)

        @plsc.parallel_loop(0, CHUNK, unroll=8)
        def row_body(r):
            for j in range(DIM // 16):
                a = buf[r, pl.ds(16 * j, 16)]
                f = buf[r, pl.ds(DIM + 16 * j, 16)]
                s, c = _sincos(f * posv)
                ob[r, pl.ds(16 * j, 16)] = a * c
                ob[r, pl.ds(DIM + 16 * j, 16)] = a * s

    def start_gather(g, idx, buf, gsem):
        pltpu.sync_copy(words_hbm.at[pl.ds(g * CHUNK, CHUNK)], idx)
        pltpu.make_async_copy(tab_hbm.at[idx], buf, gsem).start()

    def wait_gather(idx, buf, gsem):
        pltpu.make_async_copy(tab_hbm.at[idx], buf, gsem).wait()

    def out_block(g):
        q0 = g * CHUNK
        l = q0 // BATCH
        b0 = q0 % BATCH
        return out_hbm.at[pl.ds(b0, CHUNK), l], l

    start_gather(chunk0, idx0, buf0, gsem0)

    def pair_body(i, carry):
        g0 = chunk0 + 2 * i
        g1 = g0 + 1
        dst0, l0 = out_block(g0)
        dst1, l1 = out_block(g1)

        start_gather(g1, idx1, buf1, gsem1)

        wait_gather(idx0, buf0, gsem0)

        @pl.when(i > 0)
        def _():
            pltpu.make_async_copy(ob0, dst0, wsem0).wait()

        compute(buf0, ob0, l0)
        pltpu.make_async_copy(ob0, dst0, wsem0).start()

        @pl.when(i < NUM_PAIRS - 1)
        def _():
            start_gather(g0 + 2, idx0, buf0, gsem0)

        wait_gather(idx1, buf1, gsem1)

        @pl.when(i > 0)
        def _():
            pltpu.make_async_copy(ob1, dst1, wsem1).wait()

        compute(buf1, ob1, l1)
        pltpu.make_async_copy(ob1, dst1, wsem1).start()
        return carry

    lax.fori_loop(0, NUM_PAIRS, pair_body, 0)

    dstl0, _ = out_block(chunk0 + NUM_CHUNKS - 2)
    dstl1, _ = out_block(chunk0 + NUM_CHUNKS - 1)
    pltpu.make_async_copy(ob0, dstl0, wsem0).wait()
    pltpu.make_async_copy(ob1, dstl1, wsem1).wait()


@jax.jit
def _run(words_flat, tab):
    mesh = plsc.VectorSubcoreMesh(core_axis_name="c", subcore_axis_name="s")
    call = functools.partial(
        pl.kernel,
        mesh=mesh,
        compiler_params=pltpu.CompilerParams(use_tc_tiling_on_sc=False),
        out_type=jax.ShapeDtypeStruct((BATCH, SEQ, 2 * DIM), jnp.float32),
        scratch_types=[
            pltpu.VMEM((CHUNK,), jnp.int32),
            pltpu.VMEM((CHUNK,), jnp.int32),
            pltpu.VMEM((CHUNK, 2 * DIM), jnp.float32),
            pltpu.VMEM((CHUNK, 2 * DIM), jnp.float32),
            pltpu.VMEM((CHUNK, 2 * DIM), jnp.float32),
            pltpu.VMEM((CHUNK, 2 * DIM), jnp.float32),
            pltpu.SemaphoreType.DMA,
            pltpu.SemaphoreType.DMA,
            pltpu.SemaphoreType.DMA,
            pltpu.SemaphoreType.DMA,
        ],
    )(_body)
    return call(words_flat, tab)


def kernel(words, amp_table, freq_table):
    words_flat = words.T.reshape(BL).astype(jnp.int32)
    tab = jnp.concatenate([amp_table.T, freq_table.T], axis=0).T
    return _run(words_flat, tab)
